# Initial kernel scaffold; baseline (speedup 1.0000x reference)
import jax
import jax.numpy as jnp
from jax.experimental import pallas as pl


def kernel(token_ids, emb_table, W, b):
    raise NotImplementedError("write your pallas kernel here")


# speedup vs baseline: 41.0326x; 41.0326x over previous
import jax
import jax.numpy as jnp
from jax import lax
from jax.experimental import pallas as pl
from jax.experimental.pallas import tpu as pltpu
from jax.experimental.pallas import tpu_sc as plsc

B = 16384
L = 48
D = 128
V = 40
CW = 48
NW = 32
ROWS = B // NW


def _sc_hist_body(ids_hbm, counts_hbm, ids_v, counts_v):
    wid = lax.axis_index("s") * 2 + lax.axis_index("c")

    pltpu.sync_copy(ids_hbm.at[pl.ds(wid * (ROWS * L), ROWS * L)], ids_v)

    zero = jnp.zeros((16,), jnp.float32)

    def zbody(i, c):
        for j in range(16):
            counts_v[pl.ds(i * 256 + j * 16, 16)] = zero
        return c

    lax.fori_loop(0, (ROWS * CW) // 256, zbody, 0)

    lane = lax.iota(jnp.int32, 16)
    gbase0 = lane * L
    cbase0 = lane * CW
    ones = jnp.ones((16,), jnp.float32)

    def gbody(g, c):
        gb = gbase0 + g * (16 * L)
        cb = cbase0 + g * (16 * CW)
        for l in range(L):
            ids16 = plsc.load_gather(ids_v, [gb + l])
            plsc.addupdate_scatter(counts_v, [cb + ids16], ones)
        return c

    lax.fori_loop(0, ROWS // 16, gbody, 0)

    pltpu.sync_copy(counts_v, counts_hbm.at[pl.ds(wid * (ROWS * CW), ROWS * CW)])


_hist = pl.kernel(
    _sc_hist_body,
    out_type=jax.ShapeDtypeStruct((B * CW,), jnp.float32),
    mesh=plsc.VectorSubcoreMesh(
        core_axis_name="c", subcore_axis_name="s", num_cores=2, num_subcores=16
    ),
    scratch_types=[
        pltpu.VMEM((ROWS * L,), jnp.int32),
        pltpu.VMEM((ROWS * CW,), jnp.float32),
    ],
    compiler_params=pltpu.CompilerParams(needs_layout_passes=False),
)


def _tc_finish_body(counts_ref, emb_ref, w_ref, b_ref, out_ref):
    c = counts_ref[...]
    m = lax.dot_general(
        emb_ref[...], w_ref[...], (((1,), (1,)), ((), ())),
        preferred_element_type=jnp.float32,
    )
    y = lax.dot_general(
        c[:, :V], m, (((1,), (0,)), ((), ())),
        preferred_element_type=jnp.float32,
    )
    denom = jnp.maximum(jnp.float32(L) - c[:, 0:1], 1.0)
    out_ref[...] = y / denom + b_ref[...]


def kernel(token_ids, emb_table, W, b):
    counts = _hist(token_ids.reshape(-1))
    counts = counts.reshape(B, CW)
    out = pl.pallas_call(
        _tc_finish_body,
        out_shape=jax.ShapeDtypeStruct((B, D), jnp.float32),
    )(counts, emb_table, W, b.reshape(1, D))
    return out


# speedup vs baseline: 46.3317x; 1.1291x over previous
import jax
import jax.numpy as jnp
from jax import lax
from jax.experimental import pallas as pl
from jax.experimental.pallas import tpu as pltpu
from jax.experimental.pallas import tpu_sc as plsc

B = 16384
L = 48
D = 128
V = 40
CW = 48
NW = 32
ROWS = B // NW


GI = 4
SLAB_GROUPS = 8
NSLAB = (ROWS // 16) // SLAB_GROUPS


def _sc_hist_body(ids_hbm, counts_hbm, ids_v, counts_v, in_sem, out_sem):
    wid = lax.axis_index("s") * 2 + lax.axis_index("c")

    in_cp = pltpu.make_async_copy(
        ids_hbm.at[pl.ds(wid * (ROWS * L), ROWS * L)], ids_v, in_sem
    )
    in_cp.start()

    zero = jnp.zeros((16,), jnp.float32)

    def zbody(i, c):
        for j in range(16):
            counts_v[pl.ds(i * 256 + j * 16, 16)] = zero
        return c

    lax.fori_loop(0, (ROWS * CW) // 256, zbody, 0)
    in_cp.wait()

    lane = lax.iota(jnp.int32, 16)
    base0 = lane * CW
    ones = jnp.ones((16,), jnp.float32)

    def sbody(slab, c):
        def gbody(gi, c2):
            g = slab * SLAB_GROUPS + gi * GI
            bases = [base0 + (g + k) * (16 * CW) for k in range(GI)]
            for l in range(L):
                toks = [plsc.load_gather(ids_v, [bases[k] + l]) for k in range(GI)]
                for k in range(GI):
                    plsc.addupdate_scatter(counts_v, [bases[k] + toks[k]], ones)
            return c2

        lax.fori_loop(0, SLAB_GROUPS // GI, gbody, 0)
        sl = slab * (SLAB_GROUPS * 16 * CW)
        pltpu.make_async_copy(
            counts_v.at[pl.ds(sl, SLAB_GROUPS * 16 * CW)],
            counts_hbm.at[pl.ds(wid * (ROWS * CW) + sl, SLAB_GROUPS * 16 * CW)],
            out_sem,
        ).start()
        return c

    lax.fori_loop(0, NSLAB, sbody, 0)

    def dbody(slab, c):
        sl = slab * (SLAB_GROUPS * 16 * CW)
        pltpu.make_async_copy(
            counts_v.at[pl.ds(sl, SLAB_GROUPS * 16 * CW)],
            counts_hbm.at[pl.ds(wid * (ROWS * CW) + sl, SLAB_GROUPS * 16 * CW)],
            out_sem,
        ).wait()
        return c

    lax.fori_loop(0, NSLAB, dbody, 0)


_hist = pl.kernel(
    _sc_hist_body,
    out_type=jax.ShapeDtypeStruct((B * CW,), jnp.float32),
    mesh=plsc.VectorSubcoreMesh(
        core_axis_name="c", subcore_axis_name="s", num_cores=2, num_subcores=16
    ),
    scratch_types=[
        pltpu.VMEM((ROWS * L,), jnp.int32),
        pltpu.VMEM((ROWS * CW,), jnp.float32),
        pltpu.SemaphoreType.DMA,
        pltpu.SemaphoreType.DMA,
    ],
    compiler_params=pltpu.CompilerParams(needs_layout_passes=False),
)


def _tc_finish_body(counts_ref, emb_ref, w_ref, b_ref, out_ref):
    c = counts_ref[...]
    m = lax.dot_general(
        emb_ref[...], w_ref[...], (((1,), (1,)), ((), ())),
        preferred_element_type=jnp.float32,
    )
    y = lax.dot_general(
        c[:, :V], m, (((1,), (0,)), ((), ())),
        preferred_element_type=jnp.float32,
    )
    denom = jnp.maximum(jnp.float32(L) - c[:, 0:1], 1.0)
    out_ref[...] = y / denom + b_ref[...]


def kernel(token_ids, emb_table, W, b):
    counts = _hist(token_ids.reshape(-1))
    counts = counts.reshape(B, CW)
    out = pl.pallas_call(
        _tc_finish_body,
        out_shape=jax.ShapeDtypeStruct((B, D), jnp.float32),
    )(counts, emb_table, W, b.reshape(1, D))
    return out


# speedup vs baseline: 53.2073x; 1.1484x over previous
import jax
import jax.numpy as jnp
from jax import lax
from jax.experimental import pallas as pl
from jax.experimental.pallas import tpu as pltpu
from jax.experimental.pallas import tpu_sc as plsc

B = 16384
L = 48
D = 128
V = 40
CW = 128
NW = 32
ROWS = B // NW

GI = 8
SLAB_GROUPS = 8
NSLAB = (ROWS // 16) // SLAB_GROUPS


def _sc_hist_body(ids_hbm, counts_hbm, ids_v, counts_v, in_sem, out_sem):
    wid = lax.axis_index("s") * 2 + lax.axis_index("c")
    row0 = wid * ROWS

    in_cp = pltpu.make_async_copy(ids_hbm.at[pl.ds(row0, ROWS), :], ids_v, in_sem)
    in_cp.start()

    zero = jnp.zeros((16,), jnp.float32)

    def zbody(i, c):
        for j in range(8):
            for k in range(3):
                counts_v[i * 8 + j, pl.ds(k * 16, 16)] = zero
        return c

    lax.fori_loop(0, ROWS // 8, zbody, 0)
    in_cp.wait()

    lane = lax.iota(jnp.int32, 16)
    ones = jnp.ones((16,), jnp.float32)

    def sbody(slab, c):
        def gbody(gi, c2):
            g0 = slab * SLAB_GROUPS + gi * GI
            rows = [lane + (g0 + k) * 16 for k in range(GI)]
            for l in range(L):
                col = jnp.full((16,), l, jnp.int32)
                toks = [plsc.load_gather(ids_v, [rows[k], col]) for k in range(GI)]
                for k in range(GI):
                    plsc.addupdate_scatter(counts_v, [rows[k], toks[k]], ones)
            return c2

        lax.fori_loop(0, SLAB_GROUPS // GI, gbody, 0)
        slr = slab * (SLAB_GROUPS * 16)
        pltpu.make_async_copy(
            counts_v.at[pl.ds(slr, SLAB_GROUPS * 16), :],
            counts_hbm.at[pl.ds(row0 + slr, SLAB_GROUPS * 16), :],
            out_sem,
        ).start()
        return c

    lax.fori_loop(0, NSLAB, sbody, 0)

    def dbody(slab, c):
        slr = slab * (SLAB_GROUPS * 16)
        pltpu.make_async_copy(
            counts_v.at[pl.ds(slr, SLAB_GROUPS * 16), :],
            counts_hbm.at[pl.ds(row0 + slr, SLAB_GROUPS * 16), :],
            out_sem,
        ).wait()
        return c

    lax.fori_loop(0, NSLAB, dbody, 0)


_hist = pl.kernel(
    _sc_hist_body,
    out_type=jax.ShapeDtypeStruct((B, CW), jnp.float32),
    mesh=plsc.VectorSubcoreMesh(
        core_axis_name="c", subcore_axis_name="s", num_cores=2, num_subcores=16
    ),
    scratch_types=[
        pltpu.VMEM((ROWS, L), jnp.int32),
        pltpu.VMEM((ROWS, CW), jnp.float32),
        pltpu.SemaphoreType.DMA,
        pltpu.SemaphoreType.DMA,
    ],
    compiler_params=pltpu.CompilerParams(needs_layout_passes=False),
)


def _tc_finish_body(counts_ref, emb_ref, w_ref, b_ref, out_ref):
    c = counts_ref[...]
    m = lax.dot_general(
        emb_ref[...], w_ref[...], (((1,), (1,)), ((), ())),
        preferred_element_type=jnp.float32,
    )
    y = lax.dot_general(
        c[:, :V], m, (((1,), (0,)), ((), ())),
        preferred_element_type=jnp.float32,
    )
    denom = jnp.maximum(jnp.float32(L) - c[:, 0:1], 1.0)
    out_ref[...] = y / denom + b_ref[...]


def kernel(token_ids, emb_table, W, b):
    counts = _hist(token_ids)
    out = pl.pallas_call(
        _tc_finish_body,
        out_shape=jax.ShapeDtypeStruct((B, D), jnp.float32),
    )(counts, emb_table, W, b.reshape(1, D))
    return out
